# baseline (speedup 1.0000x reference)
import jax
import jax.numpy as jnp
from jax import lax
from jax.experimental import pallas as pl
from jax.experimental.pallas import tpu as pltpu
from jax.experimental.pallas import tpu_sc as plsc

_K = 2048
_ROWS = 128
_COLS = 32768
_NW = 32
_RPW = _ROWS // _NW
_NVEC = _COLS // 16
_UNROLL = 8
_CAND_CAP = 8192


def _extract0(v):
    return jnp.squeeze(lax.slice(v, (0,), (1,)))


def _extract15(v):
    return jnp.squeeze(lax.slice(v, (15,), (16,)))


def _sc_body(x_hbm, out_hbm, xb0, xb1, cand, hist, hist2, totals):
    cid = lax.axis_index("c")
    sid = lax.axis_index("s")
    wid = cid * 16 + sid
    lane = lax.iota(jnp.int32, 16)
    laneoff = lane * 256
    ones = jnp.ones((16,), jnp.int32)
    zeros16 = jnp.zeros((16,), jnp.int32)

    def keyize(v):
        sb = plsc.bitcast(v, jnp.int32)
        return jnp.where(sb < 0, sb ^ jnp.int32(0x7FFFFFFF), sb)

    def find_bucket(h, r):
        def red(g, _):
            acc = zeros16
            for bb in range(16):
                v = h[pl.ds((g * 16 + bb) * 16, 16)]
                acc = jnp.where(lane == bb, jnp.sum(v), acc)
            totals[pl.ds(g * 16, 16)] = acc
            return 0
        lax.fori_loop(0, 16, red, 0)

        z = jnp.int32(0)
        carry = (z, z, z, z, z)
        for g in range(15, -1, -1):
            S, found, bst, above, cnt = carry
            t = totals[pl.ds(g * 16, 16)]
            rv = lax.rev(t, (0,))
            cs = plsc.cumsum(rv)
            tot = cs + S
            crossed = tot >= r
            pcs = _extract0(plsc.all_reduce_population_count(crossed))
            has = pcs > 0
            pos = plsc.all_reduce_ffs(crossed)
            cs_at = jnp.max(jnp.where(lane == pos, tot, 0))
            cnt_at = jnp.max(jnp.where(lane == pos, rv, 0))
            bucket = g * 16 + 15 - _extract0(pos)
            newfound = has & (found == 0)
            bst = jnp.where(newfound, bucket, bst)
            above = jnp.where(newfound, cs_at - cnt_at, above)
            cnt = jnp.where(newfound, cnt_at, cnt)
            found = jnp.where(has, jnp.int32(1), found)
            S = _extract15(tot)
            carry = (S, found, bst, above, cnt)
        _, _, bst, above, cnt = carry
        return bst, above, cnt

    def row_threshold(xb):
        with jax.named_scope("p_clear"):
            @plsc.parallel_loop(0, 256, unroll=_UNROLL)
            def _clear(i):
                hist[pl.ds(i * 16, 16)] = zeros16
                hist2[pl.ds(i * 16, 16)] = zeros16

        with jax.named_scope("p_scan1"):
            @plsc.parallel_loop(0, _NVEC, unroll=_UNROLL)
            def _scan1(i):
                v = xb[pl.ds(i * 16, 16)]
                key = keyize(v)
                b = (key >> 24) + 128
                plsc.addupdate_scatter(hist, [b * 16 + lane], ones)

        with jax.named_scope("p_find1"):
            b1, above, _ = find_bucket(hist, jnp.int32(_K))
            r = jnp.int32(_K) - above

        with jax.named_scope("p_comp"):
            def comp(i, off):
                v = xb[pl.ds(i * 16, 16)]
                key = keyize(v)
                m = ((key >> 24) + 128) == b1
                b2v = (key >> 16) & 0xFF
                plsc.addupdate_scatter(hist2, [b2v * 16 + lane], ones, mask=m)
                plsc.store_compressed(cand.at[pl.ds(off, 16)], key, mask=m)
                return off + _extract0(plsc.all_reduce_population_count(m))
            c1 = plsc.parallel_loop(
                0, _NVEC, unroll=_UNROLL, carry=jnp.int32(0))(comp)

        with jax.named_scope("p_find2"):
            b2, above2, _ = find_bucket(hist2, r)
            r = r - above2

        with jax.named_scope("p_comp2"):
            nv1 = (c1 + 15) >> 4
            base2 = jnp.int32(_CAND_CAP)

            def comp2_fast(_):
                def body(i, off):
                    k = cand[pl.ds(i * 16, 16)]
                    valid = (i * 16 + lane) < c1
                    m = (((k >> 16) & 0xFF) == b2) & valid
                    plsc.store_compressed(
                        cand.at[pl.ds(base2 + off, 16)], k, mask=m)
                    return off + _extract0(plsc.all_reduce_population_count(m))
                c2 = plsc.parallel_loop(
                    0, _CAND_CAP // 16, unroll=_UNROLL, carry=jnp.int32(0))(body)
                return base2, c2

            def comp2_slow(_):
                def body(i, off):
                    k = cand[pl.ds(i * 16, 16)]
                    valid = (i * 16 + lane) < c1
                    m = (((k >> 16) & 0xFF) == b2) & valid
                    plsc.store_compressed(cand.at[pl.ds(off, 16)], k, mask=m)
                    return off + _extract0(plsc.all_reduce_population_count(m))
                c2 = lax.fori_loop(0, nv1, body, jnp.int32(0))
                return jnp.int32(0), c2

            start, c2 = lax.cond(c1 <= _CAND_CAP, comp2_fast, comp2_slow, 0)

        with jax.named_scope("p_bs"):
            nv2 = (c2 + 15) >> 4
            base = (((b1 - 128) << 8) | b2) << 16

            def bstep(_, lohi):
                lo, hi = lohi
                mid = lo + ((hi - lo) >> 1)

                def cbody(i, acc):
                    k = cand[pl.ds(start + i * 16, 16)]
                    valid = (i * 16 + lane) < c2
                    m = (k >= mid) & valid
                    return acc + plsc.all_reduce_population_count(m)
                cnt = _extract0(lax.fori_loop(0, nv2, cbody, zeros16))
                ge = cnt >= r
                lo = jnp.where(ge, mid, lo)
                hi = jnp.where(ge, hi, mid)
                return lo, hi
            T, _ = lax.fori_loop(
                0, 16, bstep, (base, base + jnp.int32(0x10000)))
        return T

    def row_mask(xb, T):
        with jax.named_scope("p_mask"):
            zf = jnp.zeros((16,), jnp.float32)

            @plsc.parallel_loop(0, _NVEC, unroll=_UNROLL)
            def _mask(i):
                v = xb[pl.ds(i * 16, 16)]
                key = keyize(v)
                keep = key >= T
                xb[pl.ds(i * 16, 16)] = jnp.where(
                    keep, jnp.maximum(v, 0.0), zf)

    def run(insems, outsems):
        bufs = [xb0, xb1]
        row0 = wid * _RPW
        copies = [None] * _RPW
        outs = [None] * _RPW
        copies[0] = pltpu.async_copy(x_hbm.at[row0], bufs[0], insems[0])
        copies[1] = pltpu.async_copy(x_hbm.at[row0 + 1], bufs[1], insems[1])
        for j in range(_RPW):
            xb = bufs[j % 2]
            copies[j].wait()
            T = row_threshold(xb)
            if 1 <= j <= _RPW - 2:
                outs[j - 1].wait()
                copies[j + 1] = pltpu.async_copy(
                    x_hbm.at[row0 + j + 1], bufs[(j + 1) % 2],
                    insems[(j + 1) % 2])
            row_mask(xb, T)
            outs[j] = pltpu.async_copy(
                xb, out_hbm.at[row0 + j], outsems[j % 2])
        outs[_RPW - 2].wait()
        outs[_RPW - 1].wait()

    pl.run_scoped(
        run,
        insems=[pltpu.SemaphoreType.DMA, pltpu.SemaphoreType.DMA],
        outsems=[pltpu.SemaphoreType.DMA, pltpu.SemaphoreType.DMA],
    )


@jax.jit
def kernel(x):
    mesh = plsc.VectorSubcoreMesh(
        core_axis_name="c", subcore_axis_name="s", num_cores=2, num_subcores=16)
    f = pl.kernel(
        _sc_body,
        out_type=jax.ShapeDtypeStruct((_ROWS, _COLS), jnp.float32),
        mesh=mesh,
        scratch_types=[
            pltpu.VMEM((_COLS,), jnp.float32),
            pltpu.VMEM((_COLS,), jnp.float32),
            pltpu.VMEM((_COLS + 16,), jnp.int32),
            pltpu.VMEM((4096,), jnp.int32),
            pltpu.VMEM((4096,), jnp.int32),
            pltpu.VMEM((256,), jnp.int32),
        ],
        compiler_params=pltpu.CompilerParams(needs_layout_passes=False),
    )
    return f(x)


# speedup vs baseline: 1.2164x; 1.2164x over previous
import jax
import jax.numpy as jnp
from jax import lax
from jax.experimental import pallas as pl
from jax.experimental.pallas import tpu as pltpu
from jax.experimental.pallas import tpu_sc as plsc

_K = 2048
_ROWS = 128
_COLS = 32768
_NW = 32
_RPW = _ROWS // _NW
_NVEC = _COLS // 16
_UNROLL = 8
_CAND_CAP = 8192


def _extract0(v):
    return jnp.squeeze(lax.slice(v, (0,), (1,)))


def _extract15(v):
    return jnp.squeeze(lax.slice(v, (15,), (16,)))


def _sc_body(x_hbm, out_hbm, xb0, xb1, cand, hist, hist2, totals):
    cid = lax.axis_index("c")
    sid = lax.axis_index("s")
    wid = cid * 16 + sid
    lane = lax.iota(jnp.int32, 16)
    laneoff = lane * 256
    ones = jnp.ones((16,), jnp.int32)
    zeros16 = jnp.zeros((16,), jnp.int32)

    def keyize(v):
        sb = plsc.bitcast(v, jnp.int32)
        return jnp.where(sb < 0, sb ^ jnp.int32(0x7FFFFFFF), sb)

    def find_bucket(h, r):
        def red(g, _):
            acc = zeros16
            for bb in range(16):
                v = h[pl.ds((g * 16 + bb) * 16, 16)]
                acc = jnp.where(lane == bb, jnp.sum(v), acc)
            totals[pl.ds(g * 16, 16)] = acc
            return 0
        lax.fori_loop(0, 16, red, 0)

        def body(gi, carry):
            S, found, bst, above, cnt = carry
            g = 15 - gi
            t = totals[pl.ds(g * 16, 16)]
            rv = lax.rev(t, (0,))
            cs = plsc.cumsum(rv)
            tot = cs + S
            crossed = tot >= r
            pcs = _extract0(plsc.all_reduce_population_count(crossed))
            has = pcs > 0
            pos = plsc.all_reduce_ffs(crossed)
            cs_at = jnp.max(jnp.where(lane == pos, tot, 0))
            cnt_at = jnp.max(jnp.where(lane == pos, rv, 0))
            bucket = g * 16 + 15 - _extract0(pos)
            newfound = has & (found == 0)
            bst = jnp.where(newfound, bucket, bst)
            above = jnp.where(newfound, cs_at - cnt_at, above)
            cnt = jnp.where(newfound, cnt_at, cnt)
            found = jnp.where(has, jnp.int32(1), found)
            S = _extract15(tot)
            return S, found, bst, above, cnt
        z = jnp.int32(0)
        _, _, bst, above, cnt = lax.fori_loop(0, 16, body, (z, z, z, z, z))
        return bst, above, cnt

    def row_threshold(xb):
        with jax.named_scope("p_clear"):
            @plsc.parallel_loop(0, 256, unroll=_UNROLL)
            def _clear(i):
                hist[pl.ds(i * 16, 16)] = zeros16
                hist2[pl.ds(i * 16, 16)] = zeros16

        with jax.named_scope("p_scan1"):
            @plsc.parallel_loop(0, _NVEC, unroll=_UNROLL)
            def _scan1(i):
                v = xb[pl.ds(i * 16, 16)]
                key = keyize(v)
                b = (key >> 24) + 128
                plsc.addupdate_scatter(hist, [b * 16 + lane], ones)

        with jax.named_scope("p_find1"):
            b1, above, _ = find_bucket(hist, jnp.int32(_K))
            r = jnp.int32(_K) - above

        with jax.named_scope("p_comp"):
            def comp(i, off):
                v = xb[pl.ds(i * 16, 16)]
                key = keyize(v)
                m = ((key >> 24) + 128) == b1
                b2v = (key >> 16) & 0xFF
                plsc.addupdate_scatter(hist2, [b2v * 16 + lane], ones, mask=m)
                plsc.store_compressed(cand.at[pl.ds(off, 16)], key, mask=m)
                return off + _extract0(plsc.all_reduce_population_count(m))
            c1 = plsc.parallel_loop(
                0, _NVEC, unroll=_UNROLL, carry=jnp.int32(0))(comp)

        with jax.named_scope("p_find2"):
            b2, above2, _ = find_bucket(hist2, r)
            r = r - above2

        with jax.named_scope("p_comp2"):
            nv1 = (c1 + 15) >> 4
            base2 = nv1 * 16

            def comp2_fast(_):
                def body(i, off):
                    k = cand[pl.ds(i * 16, 16)]
                    valid = (i * 16 + lane) < c1
                    m = (((k >> 16) & 0xFF) == b2) & valid
                    plsc.store_compressed(
                        cand.at[pl.ds(base2 + off, 16)], k, mask=m)
                    return off + _extract0(plsc.all_reduce_population_count(m))
                c2 = plsc.parallel_loop(
                    0, nv1, unroll=_UNROLL, carry=jnp.int32(0))(body)
                return base2, c2

            def comp2_slow(_):
                def body(i, off):
                    k = cand[pl.ds(i * 16, 16)]
                    valid = (i * 16 + lane) < c1
                    m = (((k >> 16) & 0xFF) == b2) & valid
                    plsc.store_compressed(cand.at[pl.ds(off, 16)], k, mask=m)
                    return off + _extract0(plsc.all_reduce_population_count(m))
                c2 = lax.fori_loop(0, nv1, body, jnp.int32(0))
                return jnp.int32(0), c2

            start, c2 = lax.cond(
                2 * base2 + 16 <= _COLS, comp2_fast, comp2_slow, 0)

        with jax.named_scope("p_bs"):
            nv2 = (c2 + 15) >> 4
            base = (((b1 - 128) << 8) | b2) << 16

            def bstep(_, lohi):
                lo, hi = lohi
                mid = lo + ((hi - lo) >> 1)

                def cbody(i, acc):
                    k = cand[pl.ds(start + i * 16, 16)]
                    valid = (i * 16 + lane) < c2
                    m = (k >= mid) & valid
                    return acc + plsc.all_reduce_population_count(m)
                cnt = _extract0(lax.fori_loop(0, nv2, cbody, zeros16))
                ge = cnt >= r
                lo = jnp.where(ge, mid, lo)
                hi = jnp.where(ge, hi, mid)
                return lo, hi
            T, _ = lax.fori_loop(
                0, 16, bstep, (base, base + jnp.int32(0x10000)))
        return T

    def row_mask(xb, T):
        with jax.named_scope("p_mask"):
            zf = jnp.zeros((16,), jnp.float32)

            @plsc.parallel_loop(0, _NVEC, unroll=_UNROLL)
            def _mask(i):
                v = xb[pl.ds(i * 16, 16)]
                key = keyize(v)
                keep = key >= T
                xb[pl.ds(i * 16, 16)] = jnp.where(
                    keep, jnp.maximum(v, 0.0), zf)

    def run(insems, outsems):
        bufs = [xb0, xb1]
        row0 = wid * _RPW
        copies = [None] * _RPW
        outs = [None] * _RPW
        copies[0] = pltpu.async_copy(x_hbm.at[row0], bufs[0], insems[0])
        copies[1] = pltpu.async_copy(x_hbm.at[row0 + 1], bufs[1], insems[1])
        for j in range(_RPW):
            xb = bufs[j % 2]
            copies[j].wait()
            T = row_threshold(xb)
            if 1 <= j <= _RPW - 2:
                outs[j - 1].wait()
                copies[j + 1] = pltpu.async_copy(
                    x_hbm.at[row0 + j + 1], bufs[(j + 1) % 2],
                    insems[(j + 1) % 2])
            row_mask(xb, T)
            outs[j] = pltpu.async_copy(
                xb, out_hbm.at[row0 + j], outsems[j % 2])
        outs[_RPW - 2].wait()
        outs[_RPW - 1].wait()

    pl.run_scoped(
        run,
        insems=[pltpu.SemaphoreType.DMA, pltpu.SemaphoreType.DMA],
        outsems=[pltpu.SemaphoreType.DMA, pltpu.SemaphoreType.DMA],
    )


@jax.jit
def kernel(x):
    mesh = plsc.VectorSubcoreMesh(
        core_axis_name="c", subcore_axis_name="s", num_cores=2, num_subcores=16)
    f = pl.kernel(
        _sc_body,
        out_type=jax.ShapeDtypeStruct((_ROWS, _COLS), jnp.float32),
        mesh=mesh,
        scratch_types=[
            pltpu.VMEM((_COLS,), jnp.float32),
            pltpu.VMEM((_COLS,), jnp.float32),
            pltpu.VMEM((_COLS + 16,), jnp.int32),
            pltpu.VMEM((4096,), jnp.int32),
            pltpu.VMEM((4096,), jnp.int32),
            pltpu.VMEM((256,), jnp.int32),
        ],
        compiler_params=pltpu.CompilerParams(needs_layout_passes=False),
    )
    return f(x)


# speedup vs baseline: 1.2250x; 1.0070x over previous
import jax
import jax.numpy as jnp
from jax import lax
from jax.experimental import pallas as pl
from jax.experimental.pallas import tpu as pltpu
from jax.experimental.pallas import tpu_sc as plsc

_K = 2048
_ROWS = 128
_COLS = 32768
_NW = 32
_RPW = _ROWS // _NW
_NVEC = _COLS // 16
_UNROLL = 8


def _extract0(v):
    return jnp.squeeze(lax.slice(v, (0,), (1,)))


def _extract15(v):
    return jnp.squeeze(lax.slice(v, (15,), (16,)))


def _sc_body(x_hbm, out_hbm, xb0, xb1, cand, hist, hist2, totals):
    cid = lax.axis_index("c")
    sid = lax.axis_index("s")
    wid = cid * 16 + sid
    lane = lax.iota(jnp.int32, 16)
    ones = jnp.ones((16,), jnp.int32)
    zeros16 = jnp.zeros((16,), jnp.int32)

    def keyize(v):
        sb = plsc.bitcast(v, jnp.int32)
        return jnp.where(sb < 0, sb ^ jnp.int32(0x7FFFFFFF), sb)

    def find_bucket(h, r):
        def red(g, _):
            acc = zeros16
            for bb in range(16):
                v = h[pl.ds((g * 16 + bb) * 16, 16)]
                acc = jnp.where(lane == bb, jnp.sum(v), acc)
            totals[pl.ds(g * 16, 16)] = acc
            return 0
        lax.fori_loop(0, 16, red, 0)

        def body(gi, carry):
            S, found, bst, above, cnt = carry
            g = 15 - gi
            t = totals[pl.ds(g * 16, 16)]
            rv = lax.rev(t, (0,))
            cs = plsc.cumsum(rv)
            tot = cs + S
            crossed = tot >= r
            pcs = _extract0(plsc.all_reduce_population_count(crossed))
            has = pcs > 0
            pos = plsc.all_reduce_ffs(crossed)
            cs_at = jnp.max(jnp.where(lane == pos, tot, 0))
            cnt_at = jnp.max(jnp.where(lane == pos, rv, 0))
            bucket = g * 16 + 15 - _extract0(pos)
            newfound = has & (found == 0)
            bst = jnp.where(newfound, bucket, bst)
            above = jnp.where(newfound, cs_at - cnt_at, above)
            cnt = jnp.where(newfound, cnt_at, cnt)
            found = jnp.where(has, jnp.int32(1), found)
            S = _extract15(tot)
            return S, found, bst, above, cnt
        z = jnp.int32(0)
        _, _, bst, above, cnt = lax.fori_loop(0, 16, body, (z, z, z, z, z))
        return bst, above, cnt

    def row_threshold(xb):
        @plsc.parallel_loop(0, 256, unroll=_UNROLL)
        def _clear(i):
            hist[pl.ds(i * 16, 16)] = zeros16
            hist2[pl.ds(i * 16, 16)] = zeros16

        @plsc.parallel_loop(0, _NVEC, unroll=_UNROLL)
        def _scan1(i):
            v = xb[pl.ds(i * 16, 16)]
            key = keyize(v)
            b = (key >> 24) + 128
            plsc.addupdate_scatter(hist, [b * 16 + lane], ones)

        b1, above, _ = find_bucket(hist, jnp.int32(_K))
        r = jnp.int32(_K) - above

        def comp(i, off):
            v = xb[pl.ds(i * 16, 16)]
            key = keyize(v)
            m = ((key >> 24) + 128) == b1
            b2v = (key >> 16) & 0xFF
            plsc.addupdate_scatter(hist2, [b2v * 16 + lane], ones, mask=m)
            plsc.store_compressed(cand.at[pl.ds(off, 16)], key, mask=m)
            return off + _extract0(plsc.all_reduce_population_count(m))
        c1 = plsc.parallel_loop(
            0, _NVEC, unroll=_UNROLL, carry=jnp.int32(0))(comp)

        b2, above2, _ = find_bucket(hist2, r)
        r = r - above2

        nv1 = (c1 + 15) >> 4
        base2 = nv1 * 16

        def comp2_fast(_):
            def body(i, off):
                k = cand[pl.ds(i * 16, 16)]
                valid = (i * 16 + lane) < c1
                m = (((k >> 16) & 0xFF) == b2) & valid
                plsc.store_compressed(
                    cand.at[pl.ds(base2 + off, 16)], k, mask=m)
                return off + _extract0(plsc.all_reduce_population_count(m))
            c2 = plsc.parallel_loop(
                0, nv1, unroll=_UNROLL, carry=jnp.int32(0))(body)
            return base2, c2

        def comp2_slow(_):
            def body(i, off):
                k = cand[pl.ds(i * 16, 16)]
                valid = (i * 16 + lane) < c1
                m = (((k >> 16) & 0xFF) == b2) & valid
                plsc.store_compressed(cand.at[pl.ds(off, 16)], k, mask=m)
                return off + _extract0(plsc.all_reduce_population_count(m))
            c2 = lax.fori_loop(0, nv1, body, jnp.int32(0))
            return jnp.int32(0), c2

        start, c2 = lax.cond(
            2 * base2 + 16 <= _COLS, comp2_fast, comp2_slow, 0)

        nv2 = (c2 + 15) >> 4
        base = (((b1 - 128) << 8) | b2) << 16

        def bstep(_, lohi):
            lo, hi = lohi
            mid = lo + ((hi - lo) >> 1)

            def cbody(i, acc):
                k = cand[pl.ds(start + i * 16, 16)]
                valid = (i * 16 + lane) < c2
                m = (k >= mid) & valid
                return acc + plsc.all_reduce_population_count(m)
            cnt = _extract0(lax.fori_loop(0, nv2, cbody, zeros16))
            ge = cnt >= r
            lo = jnp.where(ge, mid, lo)
            hi = jnp.where(ge, hi, mid)
            return lo, hi
        T, _ = lax.fori_loop(0, 16, bstep, (base, base + jnp.int32(0x10000)))
        return T

    def row_mask(xb, T):
        zf = jnp.zeros((16,), jnp.float32)

        @plsc.parallel_loop(0, _NVEC, unroll=_UNROLL)
        def _mask(i):
            v = xb[pl.ds(i * 16, 16)]
            key = keyize(v)
            keep = key >= T
            xb[pl.ds(i * 16, 16)] = jnp.where(keep, jnp.maximum(v, 0.0), zf)

    def run(insems, outsems):
        bufs = [xb0, xb1]
        row0 = wid * _RPW
        copies = [None] * _RPW
        outs = [None] * _RPW
        copies[0] = pltpu.async_copy(x_hbm.at[row0], bufs[0], insems[0])
        copies[1] = pltpu.async_copy(x_hbm.at[row0 + 1], bufs[1], insems[1])
        for j in range(_RPW):
            xb = bufs[j % 2]
            copies[j].wait()
            T = row_threshold(xb)
            if 1 <= j <= _RPW - 2:
                outs[j - 1].wait()
                copies[j + 1] = pltpu.async_copy(
                    x_hbm.at[row0 + j + 1], bufs[(j + 1) % 2],
                    insems[(j + 1) % 2])
            row_mask(xb, T)
            outs[j] = pltpu.async_copy(
                xb, out_hbm.at[row0 + j], outsems[j % 2])
        outs[_RPW - 2].wait()
        outs[_RPW - 1].wait()

    pl.run_scoped(
        run,
        insems=[pltpu.SemaphoreType.DMA, pltpu.SemaphoreType.DMA],
        outsems=[pltpu.SemaphoreType.DMA, pltpu.SemaphoreType.DMA],
    )


@jax.jit
def kernel(x):
    mesh = plsc.VectorSubcoreMesh(
        core_axis_name="c", subcore_axis_name="s", num_cores=2, num_subcores=16)
    f = pl.kernel(
        _sc_body,
        out_type=jax.ShapeDtypeStruct((_ROWS, _COLS), jnp.float32),
        mesh=mesh,
        scratch_types=[
            pltpu.VMEM((_COLS,), jnp.float32),
            pltpu.VMEM((_COLS,), jnp.float32),
            pltpu.VMEM((_COLS + 16,), jnp.int32),
            pltpu.VMEM((4096,), jnp.int32),
            pltpu.VMEM((4096,), jnp.int32),
            pltpu.VMEM((256,), jnp.int32),
        ],
        compiler_params=pltpu.CompilerParams(needs_layout_passes=False),
    )
    return f(x)
